# Initial kernel scaffold; baseline (speedup 1.0000x reference)
import functools

import jax
import jax.numpy as jnp
from jax import lax
from jax.experimental import pallas as pl
from jax.experimental.pallas import tpu as pltpu
from jax.experimental.pallas import tpu_sc as plsc

_TEMPERATURE = 0.5
_EPS = 1e-12

_TB = 1024
_TK = 512
_TN = 1024


def _sc_gather(table, idx):
    n_rows, d = idx.shape[0], table.shape[1]
    info = plsc.get_sparse_core_info()
    nw = info.num_cores * info.num_subcores
    b_per_w = n_rows // nw
    ch = 128
    nch = b_per_w // ch
    mesh = plsc.VectorSubcoreMesh(core_axis_name="c", subcore_axis_name="s")

    @functools.partial(
        pl.kernel,
        mesh=mesh,
        out_type=jax.ShapeDtypeStruct((n_rows, d), table.dtype),
        scratch_types=[
            pltpu.VMEM((nch, ch), jnp.int32),
            pltpu.VMEM((ch, d), table.dtype),
            pltpu.VMEM((ch, d), table.dtype),
            pltpu.SemaphoreType.DMA,
            pltpu.SemaphoreType.DMA,
        ],
    )
    def gather_kernel(table_hbm, idx_hbm, out_hbm, idx_v, rows0, rows1, sem0, sem1):
        wid = lax.axis_index("s") * info.num_cores + lax.axis_index("c")
        base = wid * b_per_w
        for j in range(nch):
            pltpu.sync_copy(idx_hbm.at[pl.ds(base + j * ch, ch)], idx_v.at[j])
        bufs = (rows0, rows1)
        sems = (sem0, sem1)
        cp = pltpu.async_copy(table_hbm.at[idx_v.at[0]], bufs[0], sems[0])
        for j in range(nch):
            cp.wait()
            if j + 1 < nch:
                cp = pltpu.async_copy(
                    table_hbm.at[idx_v.at[j + 1]], bufs[(j + 1) % 2], sems[(j + 1) % 2]
                )
            pltpu.sync_copy(bufs[j % 2], out_hbm.at[pl.ds(base + j * ch, ch)])

    return gather_kernel(table, idx)


def _norm_body(p_ref, o_ref):
    x = p_ref[...]
    n = jnp.sum(x * x, axis=1, keepdims=True)
    o_ref[...] = (x / jnp.maximum(jnp.sqrt(n), _EPS)).astype(jnp.bfloat16)


def _normalize_bf16(p):
    k, d = p.shape
    return pl.pallas_call(
        _norm_body,
        grid=(k // _TN,),
        in_specs=[pl.BlockSpec((_TN, d), lambda i: (i, 0))],
        out_specs=pl.BlockSpec((_TN, d), lambda i: (i, 0)),
        out_shape=jax.ShapeDtypeStruct((k, d), jnp.bfloat16),
    )(p)


def _fused_body(emb_ref, pg_ref, pnb_ref, out_ref, enb_s, pos_s, acc_s):
    i = pl.program_id(0)
    kk = pl.program_id(1)
    nk = pl.num_programs(1)

    @pl.when(kk == 0)
    def _prep():
        e = emb_ref[...]
        en = jnp.sum(e * e, axis=1, keepdims=True)
        es = e * ((1.0 / _TEMPERATURE) / jnp.maximum(jnp.sqrt(en), _EPS))
        enb_s[...] = es.astype(jnp.bfloat16)
        g = pg_ref[...]
        gn = jnp.sum(g * g, axis=1, keepdims=True)
        gs = g / jnp.maximum(jnp.sqrt(gn), _EPS)
        pos_s[...] = jnp.sum(es * gs, axis=1, keepdims=True)
        acc_s[...] = jnp.zeros_like(acc_s)

    s = lax.dot_general(
        enb_s[...], pnb_ref[...],
        (((1,), (1,)), ((), ())),
        preferred_element_type=jnp.float32,
    )
    p = jnp.exp(s)
    partial = p[:, 0:128]
    for j in range(1, _TK // 128):
        partial = partial + p[:, j * 128:(j + 1) * 128]
    acc_s[...] += partial

    @pl.when(kk == nk - 1)
    def _fin():
        row = jnp.sum(acc_s[...], axis=1, keepdims=True)
        pos = pos_s[...]
        tile_loss = jnp.sum(jnp.log(row - jnp.exp(pos)) - pos)

        @pl.when(i == 0)
        def _init():
            out_ref[0, 0] = tile_loss

        @pl.when(i > 0)
        def _acc():
            out_ref[0, 0] += tile_loss


def _fused_loss(emb, pg, pnb):
    b, d = emb.shape
    k = pnb.shape[0]
    total = pl.pallas_call(
        _fused_body,
        grid=(b // _TB, k // _TK),
        in_specs=[
            pl.BlockSpec((_TB, d), lambda i, kk: (i, 0)),
            pl.BlockSpec((_TB, d), lambda i, kk: (i, 0)),
            pl.BlockSpec((_TK, d), lambda i, kk: (kk, 0)),
        ],
        out_specs=pl.BlockSpec((1, 1), lambda i, kk: (0, 0)),
        out_shape=jax.ShapeDtypeStruct((1, 1), jnp.float32),
        scratch_shapes=[
            pltpu.VMEM((_TB, d), jnp.bfloat16),
            pltpu.VMEM((_TB, 1), jnp.float32),
            pltpu.VMEM((_TB, 128), jnp.float32),
        ],
        compiler_params=pltpu.CompilerParams(
            dimension_semantics=("arbitrary", "arbitrary"),
        ),
    )(emb, pg, pnb)
    return total[0, 0] / b


def kernel(embeddings, cluster_ids, prototypes):
    pg = _sc_gather(prototypes, cluster_ids.astype(jnp.int32))
    pnb = _normalize_bf16(prototypes)
    return _fused_loss(embeddings, pg, pnb)


# speedup vs baseline: 12.8231x; 12.8231x over previous
import functools

import jax
import jax.numpy as jnp
from jax import lax
from jax.experimental import pallas as pl
from jax.experimental.pallas import tpu as pltpu
from jax.experimental.pallas import tpu_sc as plsc

_TEMPERATURE = 0.5
_EPS = 1e-12

_TB = 1024
_TK = 512
_TN = 1024


def _sc_gather(table, idx):
    n_rows, d = idx.shape[0], table.shape[1]
    info = plsc.get_sparse_core_info()
    nw = info.num_cores * info.num_subcores
    b_per_w = n_rows // nw
    ch = 128
    nch = b_per_w // ch
    mesh = plsc.VectorSubcoreMesh(core_axis_name="c", subcore_axis_name="s")

    @functools.partial(
        pl.kernel,
        mesh=mesh,
        out_type=jax.ShapeDtypeStruct((n_rows, d), table.dtype),
        scratch_types=[
            pltpu.VMEM((nch, ch), jnp.int32),
            pltpu.VMEM((ch, d), table.dtype),
            pltpu.VMEM((ch, d), table.dtype),
            pltpu.SemaphoreType.DMA,
            pltpu.SemaphoreType.DMA,
        ],
    )
    def gather_kernel(table_hbm, idx_hbm, out_hbm, idx_v, rows0, rows1, sem0, sem1):
        wid = lax.axis_index("s") * info.num_cores + lax.axis_index("c")
        base = wid * b_per_w
        for j in range(nch):
            pltpu.sync_copy(idx_hbm.at[pl.ds(base + j * ch, ch)], idx_v.at[j])
        bufs = (rows0, rows1)
        sems = (sem0, sem1)
        cp = pltpu.async_copy(table_hbm.at[idx_v.at[0]], bufs[0], sems[0])
        for j in range(nch):
            cp.wait()
            if j + 1 < nch:
                cp = pltpu.async_copy(
                    table_hbm.at[idx_v.at[j + 1]], bufs[(j + 1) % 2], sems[(j + 1) % 2]
                )
            pltpu.sync_copy(bufs[j % 2], out_hbm.at[pl.ds(base + j * ch, ch)])

    return gather_kernel(table, idx)


def _norm_body(p_ref, o_ref):
    x = p_ref[...]
    n = jnp.sum(x * x, axis=1, keepdims=True)
    o_ref[...] = (x / jnp.maximum(jnp.sqrt(n), _EPS)).astype(jnp.bfloat16)


def _normalize_bf16(p):
    k, d = p.shape
    return pl.pallas_call(
        _norm_body,
        grid=(k // _TN,),
        in_specs=[pl.BlockSpec((_TN, d), lambda i: (i, 0))],
        out_specs=pl.BlockSpec((_TN, d), lambda i: (i, 0)),
        out_shape=jax.ShapeDtypeStruct((k, d), jnp.bfloat16),
    )(p)


def _fused_body(emb_ref, pg_ref, pnb_ref, out_ref, enb_s, pos_s, acc_s):
    i = pl.program_id(0)
    kk = pl.program_id(1)
    nk = pl.num_programs(1)

    @pl.when(kk == 0)
    def _prep():
        e = emb_ref[...]
        en = jnp.sum(e * e, axis=1, keepdims=True)
        es = e * ((1.0 / _TEMPERATURE) / jnp.maximum(jnp.sqrt(en), _EPS))
        enb_s[...] = es.astype(jnp.bfloat16)
        g = pg_ref[...]
        gn = jnp.sum(g * g, axis=1, keepdims=True)
        gs = g / jnp.maximum(jnp.sqrt(gn), _EPS)
        pos_s[...] = jnp.sum(es * gs, axis=1, keepdims=True)
        acc_s[...] = jnp.zeros_like(acc_s)

    s = lax.dot_general(
        enb_s[...], pnb_ref[...],
        (((1,), (1,)), ((), ())),
        preferred_element_type=jnp.float32,
    )
    p = jnp.exp(s)
    partial = p[:, 0:128]
    for j in range(1, _TK // 128):
        partial = partial + p[:, j * 128:(j + 1) * 128]
    acc_s[...] += partial

    @pl.when(kk == nk - 1)
    def _fin():
        row = jnp.sum(acc_s[...], axis=1, keepdims=True)
        pos = pos_s[...]
        tile_loss = jnp.sum(
            jnp.log(row - jnp.exp(pos)) - pos, axis=0, keepdims=True
        )

        @pl.when(i == 0)
        def _init():
            out_ref[...] = tile_loss

        @pl.when(i > 0)
        def _acc():
            out_ref[...] += tile_loss


def _fused_loss(emb, pg, pnb):
    b, d = emb.shape
    k = pnb.shape[0]
    total = pl.pallas_call(
        _fused_body,
        grid=(b // _TB, k // _TK),
        in_specs=[
            pl.BlockSpec((_TB, d), lambda i, kk: (i, 0)),
            pl.BlockSpec((_TB, d), lambda i, kk: (i, 0)),
            pl.BlockSpec((_TK, d), lambda i, kk: (kk, 0)),
        ],
        out_specs=pl.BlockSpec((1, 1), lambda i, kk: (0, 0)),
        out_shape=jax.ShapeDtypeStruct((1, 1), jnp.float32),
        scratch_shapes=[
            pltpu.VMEM((_TB, d), jnp.bfloat16),
            pltpu.VMEM((_TB, 1), jnp.float32),
            pltpu.VMEM((_TB, 128), jnp.float32),
        ],
        compiler_params=pltpu.CompilerParams(
            dimension_semantics=("arbitrary", "arbitrary"),
        ),
    )(emb, pg, pnb)
    return total[0, 0] / b


def kernel(embeddings, cluster_ids, prototypes):
    pg = _sc_gather(prototypes, cluster_ids.astype(jnp.int32))
    pnb = _normalize_bf16(prototypes)
    return _fused_loss(embeddings, pg, pnb)


# speedup vs baseline: 21.8806x; 1.7063x over previous
import functools

import jax
import jax.numpy as jnp
from jax import lax
from jax.experimental import pallas as pl
from jax.experimental.pallas import tpu as pltpu
from jax.experimental.pallas import tpu_sc as plsc

_TEMPERATURE = 0.5
_EPS = 1e-12
_LOG2E = 1.4426950408889634
_LN2 = 0.6931471805599453

_TB = 1024
_TK = 2048
_TN = 1024


def _sc_gather(table, idx):
    n_rows, d = idx.shape[0], table.shape[1]
    info = plsc.get_sparse_core_info()
    nw = info.num_cores * info.num_subcores
    b_per_w = n_rows // nw
    ch = 128
    nch = b_per_w // ch
    mesh = plsc.VectorSubcoreMesh(core_axis_name="c", subcore_axis_name="s")

    @functools.partial(
        pl.kernel,
        mesh=mesh,
        out_type=jax.ShapeDtypeStruct((n_rows, d), table.dtype),
        scratch_types=[
            pltpu.VMEM((nch, ch), jnp.int32),
            pltpu.VMEM((ch, d), table.dtype),
            pltpu.VMEM((ch, d), table.dtype),
            pltpu.SemaphoreType.DMA,
            pltpu.SemaphoreType.DMA,
        ],
    )
    def gather_kernel(table_hbm, idx_hbm, out_hbm, idx_v, rows0, rows1, sem0, sem1):
        wid = lax.axis_index("s") * info.num_cores + lax.axis_index("c")
        base = wid * b_per_w
        for j in range(nch):
            pltpu.sync_copy(idx_hbm.at[pl.ds(base + j * ch, ch)], idx_v.at[j])
        bufs = (rows0, rows1)
        sems = (sem0, sem1)
        cp = pltpu.async_copy(table_hbm.at[idx_v.at[0]], bufs[0], sems[0])
        for j in range(nch):
            cp.wait()
            if j + 1 < nch:
                cp = pltpu.async_copy(
                    table_hbm.at[idx_v.at[j + 1]], bufs[(j + 1) % 2], sems[(j + 1) % 2]
                )
            pltpu.sync_copy(bufs[j % 2], out_hbm.at[pl.ds(base + j * ch, ch)])

    return gather_kernel(table, idx)


def _norm_body(p_ref, o_ref):
    x = p_ref[...]
    n = jnp.sum(x * x, axis=1, keepdims=True)
    o_ref[...] = (x / jnp.maximum(jnp.sqrt(n), _EPS)).astype(jnp.bfloat16)


def _normalize_bf16(p):
    k, d = p.shape
    return pl.pallas_call(
        _norm_body,
        grid=(k // _TN,),
        in_specs=[pl.BlockSpec((_TN, d), lambda i: (i, 0))],
        out_specs=pl.BlockSpec((_TN, d), lambda i: (i, 0)),
        out_shape=jax.ShapeDtypeStruct((k, d), jnp.bfloat16),
    )(p)


def _fused_body(emb_ref, pg_ref, pnb_ref, out_ref, enb_s, pos_s, acc_s):
    i = pl.program_id(0)
    kk = pl.program_id(1)
    nk = pl.num_programs(1)

    @pl.when(kk == 0)
    def _prep():
        e = emb_ref[...]
        en = jnp.sum(e * e, axis=1, keepdims=True)
        es = e * ((_LOG2E / _TEMPERATURE) / jnp.maximum(jnp.sqrt(en), _EPS))
        enb_s[...] = es.astype(jnp.bfloat16)
        g = pg_ref[...]
        gn = jnp.sum(g * g, axis=1, keepdims=True)
        gs = g / jnp.maximum(jnp.sqrt(gn), _EPS)
        pos_s[...] = jnp.sum(es * gs, axis=1, keepdims=True)
        acc_s[...] = jnp.zeros_like(acc_s)

    a = enb_s[...]
    partial = acc_s[...]
    for j in range(_TK // 256):
        sj = lax.dot_general(
            a, pnb_ref[pl.ds(j * 256, 256), :],
            (((1,), (1,)), ((), ())),
            preferred_element_type=jnp.float32,
        )
        partial = partial + jnp.exp2(sj[:, 0:128]) + jnp.exp2(sj[:, 128:256])
    acc_s[...] = partial

    @pl.when(kk == nk - 1)
    def _fin():
        row = jnp.sum(acc_s[...], axis=1, keepdims=True)
        pos2 = pos_s[...]
        tile_loss = jnp.sum(
            jnp.log(row - jnp.exp2(pos2)) - pos2 * _LN2, axis=0, keepdims=True
        )

        @pl.when(i == 0)
        def _init():
            out_ref[...] = tile_loss

        @pl.when(i > 0)
        def _acc():
            out_ref[...] += tile_loss


def _fused_loss(emb, pg, pnb):
    b, d = emb.shape
    k = pnb.shape[0]
    total = pl.pallas_call(
        _fused_body,
        grid=(b // _TB, k // _TK),
        in_specs=[
            pl.BlockSpec((_TB, d), lambda i, kk: (i, 0)),
            pl.BlockSpec((_TB, d), lambda i, kk: (i, 0)),
            pl.BlockSpec((_TK, d), lambda i, kk: (kk, 0)),
        ],
        out_specs=pl.BlockSpec((1, 1), lambda i, kk: (0, 0)),
        out_shape=jax.ShapeDtypeStruct((1, 1), jnp.float32),
        scratch_shapes=[
            pltpu.VMEM((_TB, d), jnp.bfloat16),
            pltpu.VMEM((_TB, 1), jnp.float32),
            pltpu.VMEM((_TB, 128), jnp.float32),
        ],
        compiler_params=pltpu.CompilerParams(
            dimension_semantics=("arbitrary", "arbitrary"),
        ),
    )(emb, pg, pnb)
    return total[0, 0] / b


def kernel(embeddings, cluster_ids, prototypes):
    pg = _sc_gather(prototypes, cluster_ids.astype(jnp.int32))
    pnb = _normalize_bf16(prototypes)
    return _fused_loss(embeddings, pg, pnb)


# speedup vs baseline: 25.8627x; 1.1820x over previous
import functools

import jax
import jax.numpy as jnp
from jax import lax
from jax.experimental import pallas as pl
from jax.experimental.pallas import tpu as pltpu
from jax.experimental.pallas import tpu_sc as plsc

_TEMPERATURE = 0.5
_EPS = 1e-12
_LOG2E = 1.4426950408889634
_LN2 = 0.6931471805599453

_TB = 2048
_TK = 8192
_TN = 1024


def _sc_gather(table, idx):
    n_rows, d = idx.shape[0], table.shape[1]
    info = plsc.get_sparse_core_info()
    nw = info.num_cores * info.num_subcores
    b_per_w = n_rows // nw
    ch = 128
    nch = b_per_w // ch
    mesh = plsc.VectorSubcoreMesh(core_axis_name="c", subcore_axis_name="s")

    @functools.partial(
        pl.kernel,
        mesh=mesh,
        out_type=jax.ShapeDtypeStruct((n_rows, d), table.dtype),
        scratch_types=[
            pltpu.VMEM((nch, ch), jnp.int32),
            pltpu.VMEM((ch, d), table.dtype),
            pltpu.VMEM((ch, d), table.dtype),
            pltpu.SemaphoreType.DMA,
            pltpu.SemaphoreType.DMA,
        ],
    )
    def gather_kernel(table_hbm, idx_hbm, out_hbm, idx_v, rows0, rows1, sem0, sem1):
        wid = lax.axis_index("s") * info.num_cores + lax.axis_index("c")
        base = wid * b_per_w
        for j in range(nch):
            pltpu.sync_copy(idx_hbm.at[pl.ds(base + j * ch, ch)], idx_v.at[j])
        bufs = (rows0, rows1)
        sems = (sem0, sem1)
        cp = pltpu.async_copy(table_hbm.at[idx_v.at[0]], bufs[0], sems[0])
        for j in range(nch):
            cp.wait()
            if j + 1 < nch:
                cp = pltpu.async_copy(
                    table_hbm.at[idx_v.at[j + 1]], bufs[(j + 1) % 2], sems[(j + 1) % 2]
                )
            pltpu.sync_copy(bufs[j % 2], out_hbm.at[pl.ds(base + j * ch, ch)])

    return gather_kernel(table, idx)


def _fused_body(emb_ref, pg_ref, proto_ref, out_ref, pnb_s):
    i = pl.program_id(0)

    @pl.when(i == 0)
    def _prep_proto():
        p = proto_ref[...]
        pn = jnp.sum(p * p, axis=1, keepdims=True)
        pnb_s[...] = (p / jnp.maximum(jnp.sqrt(pn), _EPS)).astype(jnp.bfloat16)

    e = emb_ref[...]
    en = jnp.sum(e * e, axis=1, keepdims=True)
    es = e * ((_LOG2E / _TEMPERATURE) / jnp.maximum(jnp.sqrt(en), _EPS))
    a = es.astype(jnp.bfloat16)

    p0 = None
    p1 = None
    for j in range(pnb_s.shape[0] // 256):
        sj = lax.dot_general(
            a, pnb_s[pl.ds(j * 256, 256), :],
            (((1,), (1,)), ((), ())),
            preferred_element_type=jnp.float32,
        )
        e0 = jnp.exp2(sj[:, 0:128])
        e1 = jnp.exp2(sj[:, 128:256])
        p0 = e0 if p0 is None else p0 + e0
        p1 = e1 if p1 is None else p1 + e1

    g = pg_ref[...]
    gn = jnp.sum(g * g, axis=1, keepdims=True)
    gs = g / jnp.maximum(jnp.sqrt(gn), _EPS)
    pos2 = jnp.sum(es * gs, axis=1, keepdims=True)

    row = jnp.sum(p0 + p1, axis=1, keepdims=True)
    tile_loss = jnp.sum(
        jnp.log(row - jnp.exp2(pos2)) - pos2 * _LN2, axis=0, keepdims=True
    )

    @pl.when(i == 0)
    def _init():
        out_ref[...] = tile_loss

    @pl.when(i > 0)
    def _acc():
        out_ref[...] += tile_loss


def _fused_loss(emb, pg, proto):
    b, d = emb.shape
    k = proto.shape[0]
    total = pl.pallas_call(
        _fused_body,
        grid=(b // _TB,),
        in_specs=[
            pl.BlockSpec((_TB, d), lambda i: (i, 0)),
            pl.BlockSpec((_TB, d), lambda i: (i, 0)),
            pl.BlockSpec((k, d), lambda i: (0, 0)),
        ],
        out_specs=pl.BlockSpec((1, 1), lambda i: (0, 0)),
        out_shape=jax.ShapeDtypeStruct((1, 1), jnp.float32),
        scratch_shapes=[
            pltpu.VMEM((k, d), jnp.bfloat16),
        ],
        compiler_params=pltpu.CompilerParams(
            dimension_semantics=("arbitrary",),
        ),
    )(emb, pg, proto)
    return total[0, 0] / b


def kernel(embeddings, cluster_ids, prototypes):
    pg = _sc_gather(prototypes, cluster_ids.astype(jnp.int32))
    return _fused_loss(embeddings, pg, prototypes)


# speedup vs baseline: 26.3215x; 1.0177x over previous
import functools

import jax
import jax.numpy as jnp
from jax import lax
from jax.experimental import pallas as pl
from jax.experimental.pallas import tpu as pltpu
from jax.experimental.pallas import tpu_sc as plsc

_TEMPERATURE = 0.5
_EPS = 1e-12
_LOG2E = 1.4426950408889634
_LN2 = 0.6931471805599453

_TB = 4096
_TK = 8192
_TN = 1024


def _sc_gather(table, idx):
    n_rows, d = idx.shape[0], table.shape[1]
    info = plsc.get_sparse_core_info()
    nw = info.num_cores * info.num_subcores
    b_per_w = n_rows // nw
    ch = 128
    nch = b_per_w // ch
    mesh = plsc.VectorSubcoreMesh(core_axis_name="c", subcore_axis_name="s")

    @functools.partial(
        pl.kernel,
        mesh=mesh,
        out_type=jax.ShapeDtypeStruct((n_rows, d), table.dtype),
        scratch_types=[
            pltpu.VMEM((nch, ch), jnp.int32),
            pltpu.VMEM((ch, d), table.dtype),
            pltpu.VMEM((ch, d), table.dtype),
            pltpu.SemaphoreType.DMA,
            pltpu.SemaphoreType.DMA,
        ],
    )
    def gather_kernel(table_hbm, idx_hbm, out_hbm, idx_v, rows0, rows1, sem0, sem1):
        wid = lax.axis_index("s") * info.num_cores + lax.axis_index("c")
        base = wid * b_per_w
        for j in range(nch):
            pltpu.sync_copy(idx_hbm.at[pl.ds(base + j * ch, ch)], idx_v.at[j])
        bufs = (rows0, rows1)
        sems = (sem0, sem1)
        cp = pltpu.async_copy(table_hbm.at[idx_v.at[0]], bufs[0], sems[0])
        for j in range(nch):
            cp.wait()
            if j + 1 < nch:
                cp = pltpu.async_copy(
                    table_hbm.at[idx_v.at[j + 1]], bufs[(j + 1) % 2], sems[(j + 1) % 2]
                )
            pltpu.sync_copy(bufs[j % 2], out_hbm.at[pl.ds(base + j * ch, ch)])

    return gather_kernel(table, idx)


def _norm_body(p_ref, o_ref):
    x = p_ref[...]
    n = jnp.sum(x * x, axis=1, keepdims=True)
    o_ref[...] = (x / jnp.maximum(jnp.sqrt(n), _EPS)).astype(jnp.bfloat16)


def _normalize_bf16(p):
    k, d = p.shape
    return pl.pallas_call(
        _norm_body,
        grid=(k // _TN,),
        in_specs=[pl.BlockSpec((_TN, d), lambda i: (i, 0))],
        out_specs=pl.BlockSpec((_TN, d), lambda i: (i, 0)),
        out_shape=jax.ShapeDtypeStruct((k, d), jnp.bfloat16),
    )(p)


def _fused_body(emb_ref, pg_ref, pnb_ref, out_ref, enb_s, pos_s, acc_s):
    i = pl.program_id(0)
    kk = pl.program_id(1)
    nk = pl.num_programs(1)

    @pl.when(kk == 0)
    def _prep():
        e = emb_ref[...]
        en = jnp.sum(e * e, axis=1, keepdims=True)
        es = e * ((_LOG2E / _TEMPERATURE) / jnp.maximum(jnp.sqrt(en), _EPS))
        enb_s[...] = es.astype(jnp.bfloat16)
        g = pg_ref[...]
        gn = jnp.sum(g * g, axis=1, keepdims=True)
        gs = g / jnp.maximum(jnp.sqrt(gn), _EPS)
        pos_s[...] = jnp.sum(es * gs, axis=1, keepdims=True)
        acc_s[...] = jnp.zeros_like(acc_s)

    a = enb_s[...]
    p0 = None
    p1 = None
    for j in range(_TK // 256):
        sj = lax.dot_general(
            a, pnb_ref[pl.ds(j * 256, 256), :],
            (((1,), (1,)), ((), ())),
            preferred_element_type=jnp.float32,
        )
        e0 = jnp.exp2(sj[:, 0:128])
        e1 = jnp.exp2(sj[:, 128:256])
        p0 = e0 if p0 is None else p0 + e0
        p1 = e1 if p1 is None else p1 + e1
    acc_s[...] += p0 + p1

    @pl.when(kk == nk - 1)
    def _fin():
        row = jnp.sum(acc_s[...], axis=1, keepdims=True)
        pos2 = pos_s[...]
        tile_loss = jnp.sum(
            jnp.log(row - jnp.exp2(pos2)) - pos2 * _LN2, axis=0, keepdims=True
        )

        @pl.when(i == 0)
        def _init():
            out_ref[...] = tile_loss

        @pl.when(i > 0)
        def _acc():
            out_ref[...] += tile_loss


def _fused_loss(emb, pg, pnb):
    b, d = emb.shape
    k = pnb.shape[0]
    total = pl.pallas_call(
        _fused_body,
        grid=(b // _TB, k // _TK),
        in_specs=[
            pl.BlockSpec((_TB, d), lambda i, kk: (i, 0)),
            pl.BlockSpec((_TB, d), lambda i, kk: (i, 0)),
            pl.BlockSpec((_TK, d), lambda i, kk: (kk, 0)),
        ],
        out_specs=pl.BlockSpec((1, 1), lambda i, kk: (0, 0)),
        out_shape=jax.ShapeDtypeStruct((1, 1), jnp.float32),
        scratch_shapes=[
            pltpu.VMEM((_TB, d), jnp.bfloat16),
            pltpu.VMEM((_TB, 1), jnp.float32),
            pltpu.VMEM((_TB, 128), jnp.float32),
        ],
        compiler_params=pltpu.CompilerParams(
            dimension_semantics=("arbitrary", "arbitrary"),
        ),
    )(emb, pg, pnb)
    return total[0, 0] / b


def kernel(embeddings, cluster_ids, prototypes):
    pg = _sc_gather(prototypes, cluster_ids.astype(jnp.int32))
    pnb = _normalize_bf16(prototypes)
    return _fused_loss(embeddings, pg, pnb)


# speedup vs baseline: 26.5083x; 1.0071x over previous
import functools

import jax
import jax.numpy as jnp
from jax import lax
from jax.experimental import pallas as pl
from jax.experimental.pallas import tpu as pltpu
from jax.experimental.pallas import tpu_sc as plsc

_TEMPERATURE = 0.5
_EPS = 1e-12
_LOG2E = 1.4426950408889634
_LN2 = 0.6931471805599453

_TB = 4096
_TK = 8192
_TN = 1024


def _sc_gather(table, idx):
    n_rows, d = idx.shape[0], table.shape[1]
    info = plsc.get_sparse_core_info()
    nw = info.num_cores * info.num_subcores
    b_per_w = n_rows // nw
    ch = 128
    nch = b_per_w // ch
    mesh = plsc.VectorSubcoreMesh(core_axis_name="c", subcore_axis_name="s")

    nbuf = 3

    @functools.partial(
        pl.kernel,
        mesh=mesh,
        out_type=jax.ShapeDtypeStruct((n_rows, d), table.dtype),
        scratch_types=[
            pltpu.VMEM((nch, ch), jnp.int32),
            pltpu.VMEM((ch, d), table.dtype),
            pltpu.VMEM((ch, d), table.dtype),
            pltpu.VMEM((ch, d), table.dtype),
            pltpu.SemaphoreType.DMA,
            pltpu.SemaphoreType.DMA,
            pltpu.SemaphoreType.DMA,
            pltpu.SemaphoreType.DMA,
            pltpu.SemaphoreType.DMA,
            pltpu.SemaphoreType.DMA,
        ],
    )
    def gather_kernel(table_hbm, idx_hbm, out_hbm, idx_v,
                      rows0, rows1, rows2, gs0, gs1, gs2, ws0, ws1, ws2):
        wid = lax.axis_index("s") * info.num_cores + lax.axis_index("c")
        base = wid * b_per_w
        for j in range(nch):
            pltpu.sync_copy(idx_hbm.at[pl.ds(base + j * ch, ch)], idx_v.at[j])
        bufs = (rows0, rows1, rows2)
        gsems = (gs0, gs1, gs2)
        wsems = (ws0, ws1, ws2)

        def gather(j):
            return pltpu.async_copy(
                table_hbm.at[idx_v.at[j]], bufs[j % nbuf], gsems[j % nbuf]
            )

        gcp = [None] * nch
        wcp = [None] * nch
        for j in range(min(nbuf, nch)):
            gcp[j] = gather(j)
        for j in range(nch):
            gcp[j].wait()
            wcp[j] = pltpu.async_copy(
                bufs[j % nbuf], out_hbm.at[pl.ds(base + j * ch, ch)], wsems[j % nbuf]
            )
            nxt = j + nbuf
            if nxt < nch:
                wcp[j].wait()
                gcp[nxt] = gather(nxt)
        for j in range(max(0, nch - nbuf), nch):
            if wcp[j] is not None:
                wcp[j].wait()

    return gather_kernel(table, idx)


def _norm_body(p_ref, o_ref):
    x = p_ref[...]
    n = jnp.sum(x * x, axis=1, keepdims=True)
    o_ref[...] = (x / jnp.maximum(jnp.sqrt(n), _EPS)).astype(jnp.bfloat16)


def _normalize_bf16(p):
    k, d = p.shape
    return pl.pallas_call(
        _norm_body,
        grid=(k // _TN,),
        in_specs=[pl.BlockSpec((_TN, d), lambda i: (i, 0))],
        out_specs=pl.BlockSpec((_TN, d), lambda i: (i, 0)),
        out_shape=jax.ShapeDtypeStruct((k, d), jnp.bfloat16),
    )(p)


def _fused_body(emb_ref, pg_ref, pnb_ref, out_ref, enb_s, pos_s, acc_s):
    i = pl.program_id(0)
    kk = pl.program_id(1)
    nk = pl.num_programs(1)

    @pl.when(kk == 0)
    def _prep():
        e = emb_ref[...]
        en = jnp.sum(e * e, axis=1, keepdims=True)
        es = e * ((_LOG2E / _TEMPERATURE) / jnp.maximum(jnp.sqrt(en), _EPS))
        enb_s[...] = es.astype(jnp.bfloat16)
        g = pg_ref[...]
        gn = jnp.sum(g * g, axis=1, keepdims=True)
        gs = g / jnp.maximum(jnp.sqrt(gn), _EPS)
        pos_s[...] = jnp.sum(es * gs, axis=1, keepdims=True)
        acc_s[...] = jnp.zeros_like(acc_s)

    a = enb_s[...]
    p0 = None
    p1 = None
    for j in range(_TK // 256):
        sj = lax.dot_general(
            a, pnb_ref[pl.ds(j * 256, 256), :],
            (((1,), (1,)), ((), ())),
            preferred_element_type=jnp.float32,
        )
        e0 = jnp.exp2(sj[:, 0:128])
        e1 = jnp.exp2(sj[:, 128:256])
        p0 = e0 if p0 is None else p0 + e0
        p1 = e1 if p1 is None else p1 + e1
    acc_s[...] += p0 + p1

    @pl.when(kk == nk - 1)
    def _fin():
        row = jnp.sum(acc_s[...], axis=1, keepdims=True)
        pos2 = pos_s[...]
        tile_loss = jnp.sum(
            jnp.log(row - jnp.exp2(pos2)) - pos2 * _LN2, axis=0, keepdims=True
        )

        @pl.when(i == 0)
        def _init():
            out_ref[...] = tile_loss

        @pl.when(i > 0)
        def _acc():
            out_ref[...] += tile_loss


def _fused_loss(emb, pg, pnb):
    b, d = emb.shape
    k = pnb.shape[0]
    total = pl.pallas_call(
        _fused_body,
        grid=(b // _TB, k // _TK),
        in_specs=[
            pl.BlockSpec((_TB, d), lambda i, kk: (i, 0)),
            pl.BlockSpec((_TB, d), lambda i, kk: (i, 0)),
            pl.BlockSpec((_TK, d), lambda i, kk: (kk, 0)),
        ],
        out_specs=pl.BlockSpec((1, 1), lambda i, kk: (0, 0)),
        out_shape=jax.ShapeDtypeStruct((1, 1), jnp.float32),
        scratch_shapes=[
            pltpu.VMEM((_TB, d), jnp.bfloat16),
            pltpu.VMEM((_TB, 1), jnp.float32),
            pltpu.VMEM((_TB, 128), jnp.float32),
        ],
        compiler_params=pltpu.CompilerParams(
            dimension_semantics=("arbitrary", "arbitrary"),
        ),
    )(emb, pg, pnb)
    return total[0, 0] / b


def kernel(embeddings, cluster_ids, prototypes):
    pg = _sc_gather(prototypes, cluster_ids.astype(jnp.int32))
    pnb = _normalize_bf16(prototypes)
    return _fused_loss(embeddings, pg, pnb)
